# Initial kernel scaffold; baseline (speedup 1.0000x reference)
import jax
import jax.numpy as jnp
from jax.experimental import pallas as pl


def kernel(x, embedding_table):
    raise NotImplementedError("write your pallas kernel here")


# speedup vs baseline: 2.9631x; 2.9631x over previous
import functools

import jax
import jax.numpy as jnp
from jax import lax
from jax.experimental import pallas as pl
from jax.experimental.pallas import tpu as pltpu
from jax.experimental.pallas import tpu_sc as plsc

_NC = 2
_NS = 16
_NW = _NC * _NS
_D = 128
_C = 128


@functools.cache
def _make_gather(B):
    assert B % (_NW * _C) == 0
    n_chunks = B // (_NW * _C)
    bpw = n_chunks * _C
    mesh = plsc.VectorSubcoreMesh(core_axis_name="c", subcore_axis_name="s")

    @functools.partial(
        pl.kernel,
        out_type=jax.ShapeDtypeStruct((B, _D), jnp.float32),
        mesh=mesh,
        scratch_types=[
            pltpu.VMEM((n_chunks, _C), jnp.int32),
            pltpu.VMEM((_C, _D), jnp.float32),
            pltpu.SemaphoreType.DMA,
        ],
    )
    def body(table, idx, out, idx_v, rows, gsem):
        wid = lax.axis_index("s") * _NC + lax.axis_index("c")
        base = wid * bpw
        pltpu.sync_copy(idx.at[wid], idx_v)

        @pl.loop(0, n_chunks)
        def _(j):
            pltpu.async_copy(table.at[idx_v.at[j]], rows, gsem).wait()
            pltpu.sync_copy(rows, out.at[pl.ds(base + j * _C, _C)])

    return body


def kernel(x, embedding_table):
    B = x.size
    idx = x.reshape(_NW, B // (_NW * _C), _C)
    out = _make_gather(B)(embedding_table, idx)
    return out.reshape(x.shape + (_D,))


# speedup vs baseline: 3.3476x; 1.1298x over previous
import functools

import jax
import jax.numpy as jnp
from jax import lax
from jax.experimental import pallas as pl
from jax.experimental.pallas import tpu as pltpu
from jax.experimental.pallas import tpu_sc as plsc

_NC = 2
_NS = 16
_NW = _NC * _NS
_D = 128
_C = 128
_NBUF = 5


@functools.cache
def _make_gather(B):
    assert B % (_NW * _C) == 0
    n_chunks = B // (_NW * _C)
    assert n_chunks % _NBUF == 0 and n_chunks >= 2 * _NBUF
    bpw = n_chunks * _C
    mesh = plsc.VectorSubcoreMesh(core_axis_name="c", subcore_axis_name="s")

    @functools.partial(
        pl.kernel,
        out_type=jax.ShapeDtypeStruct((B, _D), jnp.float32),
        mesh=mesh,
        scratch_types=[
            pltpu.VMEM((n_chunks, _C), jnp.int32),
            pltpu.VMEM((_NBUF, _C, _D), jnp.float32),
            pltpu.SemaphoreType.DMA((_NBUF,)),
            pltpu.SemaphoreType.DMA((_NBUF,)),
        ],
    )
    def body(table, idx, out, idx_v, rows, gsem, wsem):
        wid = lax.axis_index("s") * _NC + lax.axis_index("c")
        base = wid * bpw
        pltpu.sync_copy(idx.at[wid], idx_v)

        for b in range(_NBUF - 1):
            pltpu.async_copy(table.at[idx_v.at[b]], rows.at[b], gsem.at[b])

        @pl.loop(0, n_chunks, step=_NBUF)
        def _(j0):
            for b in range(_NBUF):
                j = j0 + b
                bn = (b - 1) % _NBUF
                pltpu.make_async_copy(
                    table.at[idx_v.at[j]], rows.at[b], gsem.at[b]
                ).wait()
                pltpu.async_copy(
                    rows.at[b], out.at[pl.ds(base + j * _C, _C)], wsem.at[b]
                )
                jn = j + _NBUF - 1

                @pl.when(jn < n_chunks)
                def _():
                    @pl.when(j >= 1)
                    def _():
                        pltpu.make_async_copy(
                            rows.at[bn], out.at[pl.ds(base, _C)], wsem.at[bn]
                        ).wait()

                    pltpu.async_copy(
                        table.at[idx_v.at[jn]], rows.at[bn], gsem.at[bn]
                    )

        for b in range(_NBUF):
            pltpu.make_async_copy(
                rows.at[b], out.at[pl.ds(base, _C)], wsem.at[b]
            ).wait()

    return body


def kernel(x, embedding_table):
    B = x.size
    idx = x.reshape(_NW, B // (_NW * _C), _C)
    out = _make_gather(B)(embedding_table, idx)
    return out.reshape(x.shape + (_D,))
